# Initial kernel scaffold; baseline (speedup 1.0000x reference)
import jax
import jax.numpy as jnp
from jax.experimental import pallas as pl


def kernel(hidden_states):
    raise NotImplementedError("write your pallas kernel here")


# speedup vs baseline: 4.3347x; 4.3347x over previous
import functools

import jax
import jax.numpy as jnp
from jax.experimental import pallas as pl
from jax.experimental.pallas import tpu as pltpu

_B, _S, _H = 4, 8192, 2048
_CHUNK = 256
_NSTEPS = _S // _CHUNK


def _norm_bin_kernel(x_ref, out_ref, norms_ref):
    i = pl.program_id(0)
    x = x_ref[...]
    sumsq = jnp.sum(x * x, axis=-1)
    norms_ref[:, pl.ds(i * _CHUNK, _CHUNK)] = jnp.sqrt(sumsq)

    @pl.when(i == _NSTEPS - 1)
    def _finalize():
        nrm = norms_ref[...]
        dmin = jnp.min(nrm, axis=-1, keepdims=True)
        dmax = jnp.max(nrm, axis=-1, keepdims=True)
        normalized = (nrm - dmin) / (dmax - dmin + 1e-08)
        idx = (normalized * (4 - 1e-06)).astype(jnp.int32)
        idx = jnp.clip(idx, 0, 3)
        out_ref[...] = jnp.left_shift(jnp.int32(1), idx)


@jax.jit
def kernel(hidden_states):
    return pl.pallas_call(
        _norm_bin_kernel,
        grid=(_NSTEPS,),
        in_specs=[
            pl.BlockSpec((_B, _CHUNK, _H), lambda i: (0, i, 0)),
        ],
        out_specs=pl.BlockSpec((_B, _S), lambda i: (0, 0)),
        out_shape=jax.ShapeDtypeStruct((_B, _S), jnp.int32),
        scratch_shapes=[pltpu.VMEM((_B, _S), jnp.float32)],
        compiler_params=pltpu.CompilerParams(
            dimension_semantics=("arbitrary",),
        ),
    )(hidden_states)
